# baseline (speedup 1.0000x reference)
import functools

import jax
import jax.numpy as jnp
from jax import lax
from jax.experimental import pallas as pl
from jax.experimental.pallas import tpu as pltpu
from jax.experimental.pallas import tpu_sc as plsc

F32 = jnp.float32

N_NODES = 50000
N_EDGES = 800000
LATENT = 64
TWOL = 2 * LATENT

NC = 2
NS = 16
NW = NC * NS

GCH = 128
GCHUNK = N_EDGES // GCH
CH = 128
NCHUNK = N_EDGES // CH
NQ = 4
NQUART = N_NODES // NQ
TAB_ROWS = 12560
ZCH = 40
NZCH = TAB_ROWS // ZCH
WB = 50
NWB = NQUART // WB

EDGE_BLK = 3200
NODE_BLK = 2000


def _mesh():
    return plsc.VectorSubcoreMesh(core_axis_name="c", subcore_axis_name="s",
                                  num_cores=NC, num_subcores=NS)


@functools.cache
def _sc_gather_call():
    @functools.partial(
        pl.kernel,
        mesh=_mesh(),
        out_type=jax.ShapeDtypeStruct((N_EDGES, LATENT), F32),
        scratch_types=[
            pltpu.VMEM((GCH,), jnp.int32),
            pltpu.VMEM((GCH,), jnp.int32),
            pltpu.VMEM((GCH, TWOL), F32),
            pltpu.VMEM((GCH, TWOL), F32),
            pltpu.VMEM((GCH, LATENT), F32),
            pltpu.SemaphoreType.DMA,
            pltpu.SemaphoreType.DMA,
        ],
    )
    def gather_k(t_hbm, src_hbm, dst_hbm, g_hbm,
                 idx1, idx2, bufa, bufb, bufc, sem1, sem2):
        c = lax.axis_index("c")
        s = lax.axis_index("s")
        wid = s * NC + c
        n = (GCHUNK - wid + NW - 1) // NW

        def body(j, carry):
            base = (j * NW + wid) * GCH
            pltpu.sync_copy(src_hbm.at[pl.ds(base, GCH)], idx1)
            pltpu.sync_copy(dst_hbm.at[pl.ds(base, GCH)], idx2)
            cp1 = pltpu.async_copy(t_hbm.at[idx1], bufa, sem1)
            cp2 = pltpu.async_copy(t_hbm.at[idx2], bufb, sem2)
            cp1.wait()
            cp2.wait()

            def add_body(i, carry2):
                r = i // 4
                k = (i % 4) * 16
                bufc[r, pl.ds(k, 16)] = (bufa[r, pl.ds(k, 16)]
                                         + bufb[r, pl.ds(LATENT + k, 16)])
                return carry2

            lax.fori_loop(0, GCH * 4, add_body, 0, unroll=8)
            pltpu.sync_copy(bufc, g_hbm.at[pl.ds(base, GCH), :])
            return carry

        lax.fori_loop(0, n, body, 0)

    return gather_k


def _sc_gather(t, src, dst):
    return _sc_gather_call()(t, src, dst)


@functools.cache
def _sc_scatter_call():
    @functools.partial(
        pl.kernel,
        mesh=_mesh(),
        out_type=jax.ShapeDtypeStruct((N_NODES * LATENT,), F32),
        scratch_types=[
            pltpu.VMEM((CH,), jnp.int32),
            pltpu.VMEM((CH * LATENT,), F32),
            pltpu.VMEM((CH, LATENT), F32),
            pltpu.VMEM((ZCH, LATENT), F32),
            pltpu.VMEM((WB, LATENT), F32),
            pltpu.VMEM((WB * LATENT,), F32),
            pltpu.VMEM_SHARED((TAB_ROWS, LATENT), F32),
        ],
    )
    def scatter_k(he_hbm, dstc_hbm, out_hbm, idx2, datab1d, datab, zbuf,
                  wbuf, wb1d, table):
        c = lax.axis_index("c")
        s = lax.axis_index("s")

        def zfill(i, carry):
            r = i // 4
            k = (i % 4) * 16
            zbuf[r, pl.ds(k, 16)] = jnp.zeros((16,), F32)
            return carry

        lax.fori_loop(0, ZCH * 4, zfill, 0, unroll=8)

        n = (NCHUNK - s + NS - 1) // NS

        for p in range(2):
            q = c * 2 + p

            def zbody(j, carry):
                row = (j * NS + s) * ZCH
                pltpu.sync_copy(zbuf, table.at[pl.ds(row, ZCH)])
                return carry

            lax.fori_loop(0, (NZCH - s + NS - 1) // NS, zbody, 0)
            plsc.subcore_barrier()

            def body(j, carry):
                base = (j * NS + s) * CH
                pltpu.sync_copy(dstc_hbm.at[pl.ds(q * N_EDGES + base, CH)],
                                idx2)
                pltpu.sync_copy(he_hbm.at[pl.ds(base * LATENT, CH * LATENT)],
                                datab1d)

                def unpack(i, carry2):
                    datab[i // 4, pl.ds((i % 4) * 16, 16)] = \
                        datab1d[pl.ds(i * 16, 16)]
                    return carry2

                lax.fori_loop(0, CH * 4, unpack, 0, unroll=8)
                pltpu.sync_copy(datab, table.at[idx2], add=True)
                return carry

            lax.fori_loop(0, n, body, 0)
            plsc.subcore_barrier()

            def wbody(j, carry):
                row = (j * NS + s) * WB
                pltpu.sync_copy(table.at[pl.ds(row, WB)], wbuf)

                def pack(i, carry2):
                    wb1d[pl.ds(i * 16, 16)] = \
                        wbuf[i // 4, pl.ds((i % 4) * 16, 16)]
                    return carry2

                lax.fori_loop(0, WB * 4, pack, 0, unroll=8)
                pltpu.sync_copy(
                    wb1d,
                    out_hbm.at[pl.ds((q * NQUART + row) * LATENT,
                                     WB * LATENT)])
                return carry

            lax.fori_loop(0, (NWB - s + NS - 1) // NS, wbody, 0)
            plsc.subcore_barrier()

    return scatter_k


def _sc_scatter(he, dstc):
    out = _sc_scatter_call()(he.reshape(-1), dstc)
    return out.reshape(N_NODES, LATENT)


def _full(*shapes):
    return [pl.BlockSpec(s, lambda i, _s=s: tuple(0 for _ in _s))
            for s in shapes]


def _ln(o, gam, bet):
    mu = jnp.mean(o, axis=-1, keepdims=True)
    var = jnp.mean(jnp.square(o - mu), axis=-1, keepdims=True)
    return (o - mu) * lax.rsqrt(var + 1e-5) * gam + bet


def _edge_body(g, he, w1e, b1, w2, b2, gam, bet, out):
    h = g[...] + jnp.dot(he[...], w1e[...], preferred_element_type=F32) \
        + b1[...]
    h = jnp.maximum(h, 0.0)
    o = jnp.dot(h, w2[...], preferred_element_type=F32) + b2[...]
    out[...] = he[...] + _ln(o, gam[...], bet[...])


def _edge_update(g, he, w1e, b1, w2, b2, gam, bet):
    nblk = N_EDGES // EDGE_BLK
    row = pl.BlockSpec((EDGE_BLK, LATENT), lambda i: (i, 0))
    wspecs = _full((LATENT, LATENT), (1, LATENT), (LATENT, LATENT),
                   (1, LATENT), (1, LATENT), (1, LATENT))
    return pl.pallas_call(
        _edge_body,
        grid=(nblk,),
        in_specs=[row, row] + wspecs,
        out_specs=row,
        out_shape=jax.ShapeDtypeStruct((N_EDGES, LATENT), F32),
    )(g, he, w1e, b1, w2, b2, gam, bet)


def _node_common(hn, agg, wh, wa, b1, w2, b2, gam, bet):
    h = (jnp.dot(hn, wh, preferred_element_type=F32)
         + jnp.dot(agg, wa, preferred_element_type=F32) + b1)
    h = jnp.maximum(h, 0.0)
    o = jnp.dot(h, w2, preferred_element_type=F32) + b2
    return hn + _ln(o, gam, bet)


def _node_body(hn, agg, wh, wa, b1, w2, b2, gam, bet, wcat,
               hn_out, t_out):
    new = _node_common(hn[...], agg[...], wh[...], wa[...], b1[...], w2[...],
                       b2[...], gam[...], bet[...])
    hn_out[...] = new
    t_out[...] = jnp.dot(new, wcat[...], preferred_element_type=F32)


def _node_update(hn, agg, wh, wa, b1, w2, b2, gam, bet, wcat):
    nblk = N_NODES // NODE_BLK
    row = pl.BlockSpec((NODE_BLK, LATENT), lambda i: (i, 0))
    trow = pl.BlockSpec((NODE_BLK, TWOL), lambda i: (i, 0))
    wspecs = _full((LATENT, LATENT), (LATENT, LATENT), (1, LATENT),
                   (LATENT, LATENT), (1, LATENT), (1, LATENT), (1, LATENT),
                   (LATENT, TWOL))
    return pl.pallas_call(
        _node_body,
        grid=(nblk,),
        in_specs=[row, row] + wspecs,
        out_specs=(row, trow),
        out_shape=(jax.ShapeDtypeStruct((N_NODES, LATENT), F32),
                   jax.ShapeDtypeStruct((N_NODES, TWOL), F32)),
    )(hn, agg, wh, wa, b1, w2, b2, gam, bet, wcat)


def _node_last_body(hn, agg, wh, wa, b1, w2, b2, gam, bet, dw1, db1, dw2, db2,
                    y_out):
    new = _node_common(hn[...], agg[...], wh[...], wa[...], b1[...], w2[...],
                       b2[...], gam[...], bet[...])
    h = jnp.maximum(jnp.dot(new, dw1[...], preferred_element_type=F32)
                    + db1[...], 0.0)
    y_out[...] = jnp.dot(h, dw2[...], preferred_element_type=F32) + db2[...]


def _node_last(hn, agg, wh, wa, b1, w2, b2, gam, bet, dw1, db1, dw2, db2):
    nblk = N_NODES // NODE_BLK
    out_dim = dw2.shape[1]
    row = pl.BlockSpec((NODE_BLK, LATENT), lambda i: (i, 0))
    yrow = pl.BlockSpec((NODE_BLK, out_dim), lambda i: (i, 0))
    wspecs = _full((LATENT, LATENT), (LATENT, LATENT), (1, LATENT),
                   (LATENT, LATENT), (1, LATENT), (1, LATENT), (1, LATENT),
                   (LATENT, LATENT), (1, LATENT), (LATENT, out_dim),
                   (1, out_dim))
    return pl.pallas_call(
        _node_last_body,
        grid=(nblk,),
        in_specs=[row, row] + wspecs,
        out_specs=yrow,
        out_shape=jax.ShapeDtypeStruct((N_NODES, out_dim), F32),
    )(hn, agg, wh, wa, b1, w2, b2, gam, bet, dw1, db1, dw2, db2)


def _enc_node_body(x, w1, b1, w2, b2, gam, bet, wcat,
                   hn_out, t_out):
    h = jnp.maximum(jnp.dot(x[...], w1[...], preferred_element_type=F32)
                    + b1[...], 0.0)
    o = jnp.dot(h, w2[...], preferred_element_type=F32) + b2[...]
    hn = _ln(o, gam[...], bet[...])
    hn_out[...] = hn
    t_out[...] = jnp.dot(hn, wcat[...], preferred_element_type=F32)


def _enc_node(x, w1, b1, w2, b2, gam, bet, wcat):
    nblk = N_NODES // NODE_BLK
    xs = pl.BlockSpec((NODE_BLK, x.shape[1]), lambda i: (i, 0))
    row = pl.BlockSpec((NODE_BLK, LATENT), lambda i: (i, 0))
    trow = pl.BlockSpec((NODE_BLK, TWOL), lambda i: (i, 0))
    wspecs = _full((x.shape[1], LATENT), (1, LATENT), (LATENT, LATENT),
                   (1, LATENT), (1, LATENT), (1, LATENT), (LATENT, TWOL))
    return pl.pallas_call(
        _enc_node_body,
        grid=(nblk,),
        in_specs=[xs] + wspecs,
        out_specs=(row, trow),
        out_shape=(jax.ShapeDtypeStruct((N_NODES, LATENT), F32),
                   jax.ShapeDtypeStruct((N_NODES, TWOL), F32)),
    )(x, w1, b1, w2, b2, gam, bet, wcat)


def _enc_edge_body(x, w1, b1, w2, b2, gam, bet, out):
    h = jnp.maximum(jnp.dot(x[...], w1[...], preferred_element_type=F32)
                    + b1[...], 0.0)
    o = jnp.dot(h, w2[...], preferred_element_type=F32) + b2[...]
    out[...] = _ln(o, gam[...], bet[...])


def _enc_edge(x, w1, b1, w2, b2, gam, bet):
    nblk = N_EDGES // EDGE_BLK
    xs = pl.BlockSpec((EDGE_BLK, x.shape[1]), lambda i: (i, 0))
    row = pl.BlockSpec((EDGE_BLK, LATENT), lambda i: (i, 0))
    wspecs = _full((x.shape[1], LATENT), (1, LATENT), (LATENT, LATENT),
                   (1, LATENT), (1, LATENT), (1, LATENT))
    return pl.pallas_call(
        _enc_edge_body,
        grid=(nblk,),
        in_specs=[xs] + wspecs,
        out_specs=row,
        out_shape=jax.ShapeDtypeStruct((N_EDGES, LATENT), F32),
    )(x, w1, b1, w2, b2, gam, bet)


def _r(v):
    return v.reshape(1, -1)


def kernel(x, edge_index, edge_attr, params):
    assert x.shape[0] == N_NODES and edge_attr.shape[0] == N_EDGES
    src = edge_index[0].astype(jnp.int32)
    dst = edge_index[1].astype(jnp.int32)

    layers = params['layers']

    def edge_w(lyr):
        (w1, b1), (w2, b2) = lyr['edge_mlp']
        gam, bet = lyr['edge_ln']
        wcat = jnp.concatenate([w1[LATENT:2 * LATENT], w1[2 * LATENT:]],
                               axis=1)
        return (w1[:LATENT], wcat, _r(b1), w2, _r(b2), _r(gam), _r(bet))

    ne = params['node_enc']
    (nw1, nb1), (nw2, nb2) = ne['mlp']
    ngam, nbet = ne['ln']
    _, wcat0, _, _, _, _, _ = edge_w(layers[0])
    h_n, t = _enc_node(x, nw1, _r(nb1), nw2, _r(nb2), _r(ngam), _r(nbet),
                       wcat0)

    ee = params['edge_enc']
    (ew1, eb1), (ew2, eb2) = ee['mlp']
    egam, ebet = ee['ln']
    h_e = _enc_edge(edge_attr, ew1, _r(eb1), ew2, _r(eb2), _r(egam), _r(ebet))

    for li, lyr in enumerate(layers):
        w1e, _, b1, w2, b2, gam, bet = edge_w(lyr)
        g = (jnp.take(t[:, :LATENT], src, axis=0)
             + jnp.take(t[:, LATENT:], dst, axis=0))
        h_e = _edge_update(g, h_e, w1e, b1, w2, b2, gam, bet)
        agg = jax.ops.segment_sum(h_e, dst, num_segments=N_NODES)

        (mw1, mb1), (mw2, mb2) = lyr['node_mlp']
        mgam, mbet = lyr['node_ln']
        wh, wa = mw1[:LATENT], mw1[LATENT:]
        if li + 1 < len(layers):
            _, wcat_n, _, _, _, _, _ = edge_w(layers[li + 1])
            h_n, t = _node_update(h_n, agg, wh, wa, _r(mb1), mw2,
                                  _r(mb2), _r(mgam), _r(mbet), wcat_n)
        else:
            (dw1, db1), (dw2, db2) = params['decoder']
            return _node_last(h_n, agg, wh, wa, _r(mb1), mw2, _r(mb2),
                              _r(mgam), _r(mbet), dw1, _r(db1), dw2, _r(db2))
